# Initial kernel scaffold; baseline (speedup 1.0000x reference)
import jax
import jax.numpy as jnp
from jax.experimental import pallas as pl


def kernel(x, params):
    raise NotImplementedError("write your pallas kernel here")


# speedup vs baseline: 5.2097x; 5.2097x over previous
import functools

import jax
import jax.numpy as jnp
import numpy as np
from jax.experimental import pallas as pl
from jax.experimental.pallas import tpu as pltpu
from jax.experimental.pallas import tpu_sc as plsc

_NUM_CONVS = 4
_CC = 24
_G = 12
_K = 16
_B, _N = 8, 2048
_BN = _B * _N
_BNK = _BN * _K

_R = 256
_ENODES = 128
_RE = _ENODES * _K
_GW = 128

_PREC = jax.lax.Precision.HIGHEST

_SEL128 = ((np.arange(128)[:, None] % 16) == np.arange(16)[None, :]
           ).astype(np.float32)


def _prep_body(x_ref, wt_ref, bt_ref, wp_ref, bp_ref, h_ref, p_ref, *, act):
    x = x_ref[...]
    h = jax.lax.dot_general(x, wt_ref[...], (((1,), (0,)), ((), ())),
                            preferred_element_type=jnp.float32,
                            precision=_PREC) + bt_ref[...]
    if act:
        h = jnp.maximum(h, 0.0)
    h_ref[...] = h
    p_ref[...] = jax.lax.dot_general(h, wp_ref[...], (((1,), (0,)), ((), ())),
                                     preferred_element_type=jnp.float32,
                                     precision=_PREC) + bp_ref[...]


def _knn_body(hr_ref, ha_ref, idx_ref):
    b = pl.program_id(0)
    rb = pl.program_id(1)
    hr = hr_ref[0]
    ha = ha_ref[0]
    hr2 = jnp.sum(hr * hr, axis=1, keepdims=True)
    ha2 = jnp.sum(ha * ha, axis=1, keepdims=True)
    ones_r = jnp.ones_like(hr2)
    ones_a = jnp.ones_like(ha2)
    lhs = jnp.concatenate([-2.0 * hr, hr2, ones_r], axis=1)
    rhs = jnp.concatenate([ha, ones_a, ha2], axis=1)
    dist = jax.lax.dot_general(lhs, rhs, (((1,), (1,)), ((), ())),
                               preferred_element_type=jnp.float32,
                               precision=_PREC)
    iota_l = jax.lax.broadcasted_iota(jnp.int32, (_R, _N), 1)
    row_id = rb * _R + jax.lax.broadcasted_iota(jnp.int32, (_R, 1), 0)
    inf = jnp.float32(jnp.inf)
    dist = jnp.where(iota_l == row_id, inf, dist)
    cols = []
    for _ in range(_K):
        m = jnp.min(dist, axis=1, keepdims=True)
        sel = jnp.where(dist == m, iota_l, _N)
        ix = jnp.min(sel, axis=1, keepdims=True)
        cols.append(ix)
        dist = jnp.where(iota_l == ix, inf, dist)
    idx_ref[0] = jnp.concatenate(cols, axis=1) + b * _N


def _edge_body(ve_ref, im_ref, sel_ref, ue_ref, w2e_ref, w3e_ref,
               w2a_ref, w3a_ref, w3b_ref, m1_ref, m2_ref, m3_ref):
    group = jax.lax.broadcasted_iota(jnp.int32, (_RE, 128), 1) // 16
    mask = group == im_ref[...]
    vsel = jnp.where(mask, ve_ref[...], 0.0)
    vj16 = jax.lax.dot_general(vsel, sel_ref[...], (((1,), (0,)), ((), ())),
                               preferred_element_type=jnp.float32,
                               precision=_PREC)
    vj = vj16[:, :_G]
    e1 = jnp.maximum(ue_ref[...] + vj, 0.0)
    e2 = jnp.maximum(
        jax.lax.dot_general(e1, w2a_ref[...], (((1,), (0,)), ((), ())),
                            preferred_element_type=jnp.float32,
                            precision=_PREC) + w2e_ref[...], 0.0)
    e3 = (jax.lax.dot_general(e1, w3b_ref[...], (((1,), (0,)), ((), ())),
                              preferred_element_type=jnp.float32,
                              precision=_PREC)
          + jax.lax.dot_general(e2, w3a_ref[...], (((1,), (0,)), ((), ())),
                                preferred_element_type=jnp.float32,
                                precision=_PREC)
          + w3e_ref[...])
    m1_ref[...] = jnp.max(e1.reshape(_ENODES, _K, _G), axis=1)
    m2_ref[...] = jnp.max(e2.reshape(_ENODES, _K, _G), axis=1)
    m3_ref[...] = jnp.max(e3.reshape(_ENODES, _K, _G), axis=1)


def _prep_call(x, wt, bt, wp, bp, act):
    cin = x.shape[1]
    pw = wp.shape[1]
    return pl.pallas_call(
        functools.partial(_prep_body, act=act),
        grid=(1,),
        in_specs=[
            pl.BlockSpec((_BN, cin), lambda i: (0, 0)),
            pl.BlockSpec((cin, _CC), lambda i: (0, 0)),
            pl.BlockSpec((1, _CC), lambda i: (0, 0)),
            pl.BlockSpec((_CC, pw), lambda i: (0, 0)),
            pl.BlockSpec((1, pw), lambda i: (0, 0)),
        ],
        out_specs=[
            pl.BlockSpec((_BN, _CC), lambda i: (0, 0)),
            pl.BlockSpec((_BN, pw), lambda i: (0, 0)),
        ],
        out_shape=[
            jax.ShapeDtypeStruct((_BN, _CC), jnp.float32),
            jax.ShapeDtypeStruct((_BN, pw), jnp.float32),
        ],
    )(x, wt, bt.reshape(1, _CC), wp, bp.reshape(1, pw))


def _knn_call(h3):
    return pl.pallas_call(
        _knn_body,
        grid=(_B, _N // _R),
        in_specs=[
            pl.BlockSpec((1, _R, _CC), lambda b, r: (b, r, 0)),
            pl.BlockSpec((1, _N, _CC), lambda b, r: (b, 0, 0)),
        ],
        out_specs=pl.BlockSpec((1, _R, _K), lambda b, r: (b, r, 0)),
        out_shape=jax.ShapeDtypeStruct((_B, _N, _K), jnp.int32),
    )(h3, h3)


def _edge_call(ve, im, sel, ue, w2e, w3e, w2a, w3a, w3b):
    nblk = _BNK // _RE
    return pl.pallas_call(
        _edge_body,
        grid=(nblk,),
        in_specs=[
            pl.BlockSpec((_RE, 128), lambda i: (i, 0)),
            pl.BlockSpec((_RE, 1), lambda i: (i, 0)),
            pl.BlockSpec((128, 16), lambda i: (0, 0)),
            pl.BlockSpec((_RE, _G), lambda i: (i, 0)),
            pl.BlockSpec((_RE, _G), lambda i: (i, 0)),
            pl.BlockSpec((_RE, _G), lambda i: (i, 0)),
            pl.BlockSpec((_G, _G), lambda i: (0, 0)),
            pl.BlockSpec((_G, _G), lambda i: (0, 0)),
            pl.BlockSpec((_G, _G), lambda i: (0, 0)),
        ],
        out_specs=[
            pl.BlockSpec((_ENODES, _G), lambda i: (i, 0)),
            pl.BlockSpec((_ENODES, _G), lambda i: (i, 0)),
            pl.BlockSpec((_ENODES, _G), lambda i: (i, 0)),
        ],
        out_shape=[
            jax.ShapeDtypeStruct((_BN, _G), jnp.float32),
            jax.ShapeDtypeStruct((_BN, _G), jnp.float32),
            jax.ShapeDtypeStruct((_BN, _G), jnp.float32),
        ],
    )(ve, im, sel, ue, w2e, w3e, w2a, w3a, w3b)


def _sc_gather(vpack, idx_flat):
    mesh = plsc.VectorSubcoreMesh(core_axis_name="c", subcore_axis_name="s")

    @pl.kernel(out_type=jax.ShapeDtypeStruct((_BNK, 128), jnp.float32),
               mesh=mesh)
    def gk(x_hbm, i_hbm, o_hbm):
        def body(i_vmem, o_vmem):
            pltpu.sync_copy(x_hbm.at[i_vmem.at[0]], o_vmem)

        pltpu.emit_pipeline(
            body,
            grid=(_BNK // _GW,),
            in_specs=[pl.BlockSpec((1, _GW), index_map=lambda i: (0, i))],
            out_specs=[pl.BlockSpec((_GW, 128), index_map=lambda i: (i, 0))],
            core_axis_name=("c", "s"),
            dimension_semantics=(pltpu.PARALLEL,),
        )(i_hbm, o_hbm)

    return gk(vpack, idx_flat)


def _layer_weights(params, i):
    w1 = params[f"conv{i}_first_W"]
    b1 = params[f"conv{i}_first_b"]
    if i == 0:
        wu, wv = -w1, w1
    else:
        wu = w1[0:_CC] - w1[2 * _CC:3 * _CC]
        wv = w1[_CC:2 * _CC] + w1[2 * _CC:3 * _CC]
    w2 = params[f"conv{i}_mid0_W"]
    b2 = params[f"conv{i}_mid0_b"]
    w3 = params[f"conv{i}_last_W"]
    b3 = params[f"conv{i}_last_b"]
    w2a, w2b = w2[:_G], w2[_G:]
    w3a, w3b, w3c = w3[:_G], w3[_G:2 * _G], w3[2 * _G:]
    wv16 = jnp.pad(wv, ((0, 0), (0, 16 - _G)))
    wp = jnp.concatenate([wu, wv16, w2b, w3c], axis=1)
    bp = jnp.concatenate([b1, jnp.zeros((16,), jnp.float32), b2, b3])
    return wp, bp, w2a, w3a, w3b


def kernel(x, params):
    xf = x.reshape(_BN, -1)
    for i in range(_NUM_CONVS):
        wt = params[f"trans{i}_W"]
        bt = params[f"trans{i}_b"]
        wp, bp, w2a, w3a, w3b = _layer_weights(params, i)
        h, p = _prep_call(xf, wt, bt, wp, bp, act=(i != 0))
        idx = _knn_call(h.reshape(_B, _N, _CC))
        vpack = p[:, _G:_G + 16].reshape(_BN // 8, 128)
        idxf = idx.reshape(_BNK)
        ve = _sc_gather(vpack, (idxf // 8).reshape(1, _BNK))
        im = (idxf % 8).astype(jnp.int32).reshape(_BNK, 1)
        sel = jnp.asarray(_SEL128)
        ue = jnp.repeat(p[:, :_G], _K, axis=0)
        w2e = jnp.repeat(p[:, _G + 16:2 * _G + 16], _K, axis=0)
        w3e = jnp.repeat(p[:, 2 * _G + 16:], _K, axis=0)
        m1, m2, m3 = _edge_call(ve, im, sel, ue, w2e, w3e, w2a, w3a, w3b)
        xf = jnp.concatenate([m3, m2, m1, h], axis=1)
    return xf.reshape(_B, _N, 3 * _G + _CC)


# speedup vs baseline: 5.8210x; 1.1173x over previous
import functools

import jax
import jax.numpy as jnp
import numpy as np
from jax.experimental import pallas as pl
from jax.experimental.pallas import tpu as pltpu
from jax.experimental.pallas import tpu_sc as plsc

_NUM_CONVS = 4
_CC = 24
_G = 12
_K = 16
_B, _N = 8, 2048
_BN = _B * _N
_BNK = _BN * _K

_R = 256
_ENODES = 128
_RE = _ENODES * _K
_GW = 128

_PREC = jax.lax.Precision.HIGHEST

_SEL128 = ((np.arange(128)[:, None] % 16) == np.arange(16)[None, :]
           ).astype(np.float32)


def _prep_body(x_ref, wt_ref, bt_ref, wp_ref, bp_ref, h_ref, p_ref, *, act):
    x = x_ref[...]
    h = jax.lax.dot_general(x, wt_ref[...], (((1,), (0,)), ((), ())),
                            preferred_element_type=jnp.float32,
                            precision=_PREC) + bt_ref[...]
    if act:
        h = jnp.maximum(h, 0.0)
    h_ref[...] = h
    p_ref[...] = jax.lax.dot_general(h, wp_ref[...], (((1,), (0,)), ((), ())),
                                     preferred_element_type=jnp.float32,
                                     precision=_PREC) + bp_ref[...]


def _knn_body(hr_ref, ha_ref, idx_ref):
    b = pl.program_id(0)
    rb = pl.program_id(1)
    hr = hr_ref[0]
    ha = ha_ref[0]
    hr2 = jnp.sum(hr * hr, axis=1, keepdims=True)
    ha2 = jnp.sum(ha * ha, axis=1, keepdims=True)
    ones_r = jnp.ones_like(hr2)
    ones_a = jnp.ones_like(ha2)
    lhs = jnp.concatenate([-2.0 * hr, hr2, ones_r], axis=1)
    rhs = jnp.concatenate([ha, ones_a, ha2], axis=1)
    dist = jax.lax.dot_general(lhs, rhs, (((1,), (1,)), ((), ())),
                               preferred_element_type=jnp.float32,
                               precision=_PREC)
    iota_l = jax.lax.broadcasted_iota(jnp.int32, (_R, _N), 1)
    row_id = rb * _R + jax.lax.broadcasted_iota(jnp.int32, (_R, 1), 0)
    dist = jnp.maximum(dist, 0.0)
    keys = (jax.lax.bitcast_convert_type(dist, jnp.int32) & ~2047) | iota_l
    big = jnp.int32(0x7FFFFFFF)
    keys = jnp.where(iota_l == row_id, big, keys)
    cols = []
    for _ in range(_K):
        m = jnp.min(keys, axis=1, keepdims=True)
        cols.append(m & 2047)
        keys = jnp.where(keys == m, big, keys)
    idx_ref[0] = jnp.concatenate(cols, axis=1) + b * _N


def _edge_body(ve_ref, im_ref, sel_ref, ue_ref, w2e_ref, w3e_ref,
               w2a_ref, w3a_ref, w3b_ref, m1_ref, m2_ref, m3_ref):
    group = jax.lax.broadcasted_iota(jnp.int32, (_RE, 128), 1) // 16
    mask = group == im_ref[...]
    vsel = jnp.where(mask, ve_ref[...], 0.0)
    vj16 = jax.lax.dot_general(vsel, sel_ref[...], (((1,), (0,)), ((), ())),
                               preferred_element_type=jnp.float32,
                               precision=_PREC)
    vj = vj16[:, :_G]
    e1 = jnp.maximum(ue_ref[...] + vj, 0.0)
    e2 = jnp.maximum(
        jax.lax.dot_general(e1, w2a_ref[...], (((1,), (0,)), ((), ())),
                            preferred_element_type=jnp.float32,
                            precision=_PREC) + w2e_ref[...], 0.0)
    e3 = (jax.lax.dot_general(e1, w3b_ref[...], (((1,), (0,)), ((), ())),
                              preferred_element_type=jnp.float32,
                              precision=_PREC)
          + jax.lax.dot_general(e2, w3a_ref[...], (((1,), (0,)), ((), ())),
                                preferred_element_type=jnp.float32,
                                precision=_PREC)
          + w3e_ref[...])
    m1_ref[...] = jnp.max(e1.reshape(_ENODES, _K, _G), axis=1)
    m2_ref[...] = jnp.max(e2.reshape(_ENODES, _K, _G), axis=1)
    m3_ref[...] = jnp.max(e3.reshape(_ENODES, _K, _G), axis=1)


def _prep_call(x, wt, bt, wp, bp, act):
    cin = x.shape[1]
    pw = wp.shape[1]
    return pl.pallas_call(
        functools.partial(_prep_body, act=act),
        grid=(1,),
        in_specs=[
            pl.BlockSpec((_BN, cin), lambda i: (0, 0)),
            pl.BlockSpec((cin, _CC), lambda i: (0, 0)),
            pl.BlockSpec((1, _CC), lambda i: (0, 0)),
            pl.BlockSpec((_CC, pw), lambda i: (0, 0)),
            pl.BlockSpec((1, pw), lambda i: (0, 0)),
        ],
        out_specs=[
            pl.BlockSpec((_BN, _CC), lambda i: (0, 0)),
            pl.BlockSpec((_BN, pw), lambda i: (0, 0)),
        ],
        out_shape=[
            jax.ShapeDtypeStruct((_BN, _CC), jnp.float32),
            jax.ShapeDtypeStruct((_BN, pw), jnp.float32),
        ],
    )(x, wt, bt.reshape(1, _CC), wp, bp.reshape(1, pw))


def _knn_call(h3):
    return pl.pallas_call(
        _knn_body,
        grid=(_B, _N // _R),
        in_specs=[
            pl.BlockSpec((1, _R, _CC), lambda b, r: (b, r, 0)),
            pl.BlockSpec((1, _N, _CC), lambda b, r: (b, 0, 0)),
        ],
        out_specs=pl.BlockSpec((1, _R, _K), lambda b, r: (b, r, 0)),
        out_shape=jax.ShapeDtypeStruct((_B, _N, _K), jnp.int32),
    )(h3, h3)


def _edge_call(ve, im, sel, ue, w2e, w3e, w2a, w3a, w3b):
    nblk = _BNK // _RE
    return pl.pallas_call(
        _edge_body,
        grid=(nblk,),
        in_specs=[
            pl.BlockSpec((_RE, 128), lambda i: (i, 0)),
            pl.BlockSpec((_RE, 1), lambda i: (i, 0)),
            pl.BlockSpec((128, 16), lambda i: (0, 0)),
            pl.BlockSpec((_RE, _G), lambda i: (i, 0)),
            pl.BlockSpec((_RE, _G), lambda i: (i, 0)),
            pl.BlockSpec((_RE, _G), lambda i: (i, 0)),
            pl.BlockSpec((_G, _G), lambda i: (0, 0)),
            pl.BlockSpec((_G, _G), lambda i: (0, 0)),
            pl.BlockSpec((_G, _G), lambda i: (0, 0)),
        ],
        out_specs=[
            pl.BlockSpec((_ENODES, _G), lambda i: (i, 0)),
            pl.BlockSpec((_ENODES, _G), lambda i: (i, 0)),
            pl.BlockSpec((_ENODES, _G), lambda i: (i, 0)),
        ],
        out_shape=[
            jax.ShapeDtypeStruct((_BN, _G), jnp.float32),
            jax.ShapeDtypeStruct((_BN, _G), jnp.float32),
            jax.ShapeDtypeStruct((_BN, _G), jnp.float32),
        ],
    )(ve, im, sel, ue, w2e, w3e, w2a, w3a, w3b)


def _sc_gather(vpack, idx_flat):
    mesh = plsc.VectorSubcoreMesh(core_axis_name="c", subcore_axis_name="s")

    @pl.kernel(out_type=jax.ShapeDtypeStruct((_BNK, 128), jnp.float32),
               mesh=mesh)
    def gk(x_hbm, i_hbm, o_hbm):
        def body(i_vmem, o_vmem):
            pltpu.sync_copy(x_hbm.at[i_vmem.at[0]], o_vmem)

        pltpu.emit_pipeline(
            body,
            grid=(_BNK // _GW,),
            in_specs=[pl.BlockSpec((1, _GW), index_map=lambda i: (0, i))],
            out_specs=[pl.BlockSpec((_GW, 128), index_map=lambda i: (i, 0))],
            core_axis_name=("c", "s"),
            dimension_semantics=(pltpu.PARALLEL,),
        )(i_hbm, o_hbm)

    return gk(vpack, idx_flat)


def _layer_weights(params, i):
    w1 = params[f"conv{i}_first_W"]
    b1 = params[f"conv{i}_first_b"]
    if i == 0:
        wu, wv = -w1, w1
    else:
        wu = w1[0:_CC] - w1[2 * _CC:3 * _CC]
        wv = w1[_CC:2 * _CC] + w1[2 * _CC:3 * _CC]
    w2 = params[f"conv{i}_mid0_W"]
    b2 = params[f"conv{i}_mid0_b"]
    w3 = params[f"conv{i}_last_W"]
    b3 = params[f"conv{i}_last_b"]
    w2a, w2b = w2[:_G], w2[_G:]
    w3a, w3b, w3c = w3[:_G], w3[_G:2 * _G], w3[2 * _G:]
    wv16 = jnp.pad(wv, ((0, 0), (0, 16 - _G)))
    wp = jnp.concatenate([wu, wv16, w2b, w3c], axis=1)
    bp = jnp.concatenate([b1, jnp.zeros((16,), jnp.float32), b2, b3])
    return wp, bp, w2a, w3a, w3b


def kernel(x, params):
    xf = x.reshape(_BN, -1)
    for i in range(_NUM_CONVS):
        wt = params[f"trans{i}_W"]
        bt = params[f"trans{i}_b"]
        wp, bp, w2a, w3a, w3b = _layer_weights(params, i)
        h, p = _prep_call(xf, wt, bt, wp, bp, act=(i != 0))
        idx = _knn_call(h.reshape(_B, _N, _CC))
        vpack = p[:, _G:_G + 16].reshape(_BN // 8, 128)
        idxf = idx.reshape(_BNK)
        ve = _sc_gather(vpack, (idxf // 8).reshape(1, _BNK))
        im = (idxf % 8).astype(jnp.int32).reshape(_BNK, 1)
        sel = jnp.asarray(_SEL128)
        ue = jnp.repeat(p[:, :_G], _K, axis=0)
        w2e = jnp.repeat(p[:, _G + 16:2 * _G + 16], _K, axis=0)
        w3e = jnp.repeat(p[:, 2 * _G + 16:], _K, axis=0)
        m1, m2, m3 = _edge_call(ve, im, sel, ue, w2e, w3e, w2a, w3a, w3b)
        xf = jnp.concatenate([m3, m2, m1, h], axis=1)
    return xf.reshape(_B, _N, 3 * _G + _CC)


# speedup vs baseline: 6.2812x; 1.0791x over previous
import functools

import jax
import jax.numpy as jnp
import numpy as np
from jax.experimental import pallas as pl
from jax.experimental.pallas import tpu as pltpu
from jax.experimental.pallas import tpu_sc as plsc

_NUM_CONVS = 4
_CC = 24
_G = 12
_K = 16
_B, _N = 8, 2048
_BN = _B * _N
_BNK = _BN * _K

_R = 256
_ENODES = 128
_RE = _ENODES * _K
_GW = 128

_PREC = jax.lax.Precision.HIGHEST

_SEL128 = ((np.arange(128)[:, None] % 16) == np.arange(16)[None, :]
           ).astype(np.float32)


def _prep_body(x_ref, wt_ref, bt_ref, wp_ref, bp_ref, h_ref, p_ref, *, act):
    x = x_ref[...]
    h = jax.lax.dot_general(x, wt_ref[...], (((1,), (0,)), ((), ())),
                            preferred_element_type=jnp.float32,
                            precision=_PREC) + bt_ref[...]
    if act:
        h = jnp.maximum(h, 0.0)
    h_ref[...] = h
    p_ref[...] = jax.lax.dot_general(h, wp_ref[...], (((1,), (0,)), ((), ())),
                                     preferred_element_type=jnp.float32,
                                     precision=_PREC) + bp_ref[...]


def _knn_body(hr_ref, ha_ref, idx_ref, *, b0):
    b = pl.program_id(0) + b0
    rb = pl.program_id(1)
    hr = hr_ref[0]
    ha = ha_ref[0]
    hr2 = jnp.sum(hr * hr, axis=1, keepdims=True)
    ha2 = jnp.sum(ha * ha, axis=1, keepdims=True)
    ones_r = jnp.ones_like(hr2)
    ones_a = jnp.ones_like(ha2)
    lhs = jnp.concatenate([-2.0 * hr, hr2, ones_r], axis=1)
    rhs = jnp.concatenate([ha, ones_a, ha2], axis=1)
    dist = jax.lax.dot_general(lhs, rhs, (((1,), (1,)), ((), ())),
                               preferred_element_type=jnp.float32,
                               precision=_PREC)
    iota_l = jax.lax.broadcasted_iota(jnp.int32, (_R, _N), 1)
    row_id = rb * _R + jax.lax.broadcasted_iota(jnp.int32, (_R, 1), 0)
    dist = jnp.maximum(dist, 0.0)
    keys = (jax.lax.bitcast_convert_type(dist, jnp.int32) & ~2047) | iota_l
    big = jnp.int32(0x7FFFFFFF)
    keys = jnp.where(iota_l == row_id, big, keys)
    cols = []
    for _ in range(_K):
        m = jnp.min(keys, axis=1, keepdims=True)
        cols.append(m & 2047)
        keys = jnp.where(keys == m, big, keys)
    idx_ref[0] = jnp.concatenate(cols, axis=1) + b * _N


def _bcast_nodes(a_ref):
    a = a_ref[...]
    return jnp.broadcast_to(a[:, None, :], (_ENODES, _K, _G)).reshape(_RE, _G)


def _edge_body(ve_ref, im_ref, sel_ref, u_ref, w2_ref, w3_ref,
               w2a_ref, w3a_ref, w3b_ref, m1_ref, m2_ref, m3_ref):
    group = jax.lax.broadcasted_iota(jnp.int32, (_RE, 128), 1) // 16
    mask = group == im_ref[...]
    vsel = jnp.where(mask, ve_ref[...], 0.0)
    vj16 = jax.lax.dot_general(vsel, sel_ref[...], (((1,), (0,)), ((), ())),
                               preferred_element_type=jnp.float32,
                               precision=_PREC)
    vj = vj16[:, :_G]
    e1 = jnp.maximum(_bcast_nodes(u_ref) + vj, 0.0)
    e2 = jnp.maximum(
        jax.lax.dot_general(e1, w2a_ref[...], (((1,), (0,)), ((), ())),
                            preferred_element_type=jnp.float32,
                            precision=_PREC) + _bcast_nodes(w2_ref), 0.0)
    e3 = (jax.lax.dot_general(e1, w3b_ref[...], (((1,), (0,)), ((), ())),
                              preferred_element_type=jnp.float32,
                              precision=_PREC)
          + jax.lax.dot_general(e2, w3a_ref[...], (((1,), (0,)), ((), ())),
                                preferred_element_type=jnp.float32,
                                precision=_PREC)
          + _bcast_nodes(w3_ref))
    m1_ref[...] = jnp.max(e1.reshape(_ENODES, _K, _G), axis=1)
    m2_ref[...] = jnp.max(e2.reshape(_ENODES, _K, _G), axis=1)
    m3_ref[...] = jnp.max(e3.reshape(_ENODES, _K, _G), axis=1)


def _prep_call(x, wt, bt, wp, bp, act):
    cin = x.shape[1]
    pw = wp.shape[1]
    return pl.pallas_call(
        functools.partial(_prep_body, act=act),
        grid=(1,),
        in_specs=[
            pl.BlockSpec((_BN, cin), lambda i: (0, 0)),
            pl.BlockSpec((cin, _CC), lambda i: (0, 0)),
            pl.BlockSpec((1, _CC), lambda i: (0, 0)),
            pl.BlockSpec((_CC, pw), lambda i: (0, 0)),
            pl.BlockSpec((1, pw), lambda i: (0, 0)),
        ],
        out_specs=[
            pl.BlockSpec((_BN, _CC), lambda i: (0, 0)),
            pl.BlockSpec((_BN, pw), lambda i: (0, 0)),
        ],
        out_shape=[
            jax.ShapeDtypeStruct((_BN, _CC), jnp.float32),
            jax.ShapeDtypeStruct((_BN, pw), jnp.float32),
        ],
    )(x, wt, bt.reshape(1, _CC), wp, bp.reshape(1, pw))


def _knn_call(h3, b0):
    nb = h3.shape[0]
    return pl.pallas_call(
        functools.partial(_knn_body, b0=b0),
        grid=(nb, _N // _R),
        in_specs=[
            pl.BlockSpec((1, _R, _CC), lambda b, r: (b, r, 0)),
            pl.BlockSpec((1, _N, _CC), lambda b, r: (b, 0, 0)),
        ],
        out_specs=pl.BlockSpec((1, _R, _K), lambda b, r: (b, r, 0)),
        out_shape=jax.ShapeDtypeStruct((nb, _N, _K), jnp.int32),
    )(h3, h3)


def _edge_call(ve, im, sel, u, w2, w3, w2a, w3a, w3b):
    n_nodes = u.shape[0]
    nblk = n_nodes // _ENODES
    return pl.pallas_call(
        _edge_body,
        grid=(nblk,),
        in_specs=[
            pl.BlockSpec((_RE, 128), lambda i: (i, 0)),
            pl.BlockSpec((_RE, 1), lambda i: (i, 0)),
            pl.BlockSpec((128, 16), lambda i: (0, 0)),
            pl.BlockSpec((_ENODES, _G), lambda i: (i, 0)),
            pl.BlockSpec((_ENODES, _G), lambda i: (i, 0)),
            pl.BlockSpec((_ENODES, _G), lambda i: (i, 0)),
            pl.BlockSpec((_G, _G), lambda i: (0, 0)),
            pl.BlockSpec((_G, _G), lambda i: (0, 0)),
            pl.BlockSpec((_G, _G), lambda i: (0, 0)),
        ],
        out_specs=[
            pl.BlockSpec((_ENODES, _G), lambda i: (i, 0)),
            pl.BlockSpec((_ENODES, _G), lambda i: (i, 0)),
            pl.BlockSpec((_ENODES, _G), lambda i: (i, 0)),
        ],
        out_shape=[
            jax.ShapeDtypeStruct((n_nodes, _G), jnp.float32),
            jax.ShapeDtypeStruct((n_nodes, _G), jnp.float32),
            jax.ShapeDtypeStruct((n_nodes, _G), jnp.float32),
        ],
    )(ve, im, sel, u, w2, w3, w2a, w3a, w3b)


def _sc_gather(vpack, idx_flat):
    mesh = plsc.VectorSubcoreMesh(core_axis_name="c", subcore_axis_name="s")
    nidx = idx_flat.shape[1]

    @pl.kernel(out_type=jax.ShapeDtypeStruct((nidx, 128), jnp.float32),
               mesh=mesh)
    def gk(x_hbm, i_hbm, o_hbm):
        def body(i_vmem, o_vmem):
            pltpu.sync_copy(x_hbm.at[i_vmem.at[0]], o_vmem)

        pltpu.emit_pipeline(
            body,
            grid=(nidx // _GW,),
            in_specs=[pl.BlockSpec((1, _GW), index_map=lambda i: (0, i))],
            out_specs=[pl.BlockSpec((_GW, 128), index_map=lambda i: (i, 0))],
            core_axis_name=("c", "s"),
            dimension_semantics=(pltpu.PARALLEL,),
        )(i_hbm, o_hbm)

    return gk(vpack, idx_flat)


def _layer_weights(params, i):
    w1 = params[f"conv{i}_first_W"]
    b1 = params[f"conv{i}_first_b"]
    if i == 0:
        wu, wv = -w1, w1
    else:
        wu = w1[0:_CC] - w1[2 * _CC:3 * _CC]
        wv = w1[_CC:2 * _CC] + w1[2 * _CC:3 * _CC]
    w2 = params[f"conv{i}_mid0_W"]
    b2 = params[f"conv{i}_mid0_b"]
    w3 = params[f"conv{i}_last_W"]
    b3 = params[f"conv{i}_last_b"]
    w2a, w2b = w2[:_G], w2[_G:]
    w3a, w3b, w3c = w3[:_G], w3[_G:2 * _G], w3[2 * _G:]
    wv16 = jnp.pad(wv, ((0, 0), (0, 16 - _G)))
    wp = jnp.concatenate([wu, wv16, w2b, w3c], axis=1)
    bp = jnp.concatenate([b1, jnp.zeros((16,), jnp.float32), b2, b3])
    return wp, bp, w2a, w3a, w3b


def kernel(x, params):
    xf = x.reshape(_BN, -1)
    for i in range(_NUM_CONVS):
        wt = params[f"trans{i}_W"]
        bt = params[f"trans{i}_b"]
        wp, bp, w2a, w3a, w3b = _layer_weights(params, i)
        h, p = _prep_call(xf, wt, bt, wp, bp, act=(i != 0))
        h3 = h.reshape(_B, _N, _CC)
        vpack = p[:, _G:_G + 16].reshape(_BN // 8, 128)
        sel = jnp.asarray(_SEL128)
        bh = _B // 2
        nh = bh * _N
        outs = []
        for half in range(2):
            idx = _knn_call(h3[half * bh:(half + 1) * bh], half * bh)
            idxf = idx.reshape(nh * _K)
            ve = _sc_gather(vpack, (idxf // 8).reshape(1, nh * _K))
            im = (idxf % 8).astype(jnp.int32).reshape(nh * _K, 1)
            sl = slice(half * nh, (half + 1) * nh)
            m1, m2, m3 = _edge_call(ve, im, sel, p[sl, :_G],
                                    p[sl, _G + 16:2 * _G + 16],
                                    p[sl, 2 * _G + 16:], w2a, w3a, w3b)
            outs.append(jnp.concatenate([m3, m2, m1, h[sl]], axis=1))
        xf = jnp.concatenate(outs, axis=0)
    return xf.reshape(_B, _N, 3 * _G + _CC)
